# Initial kernel scaffold; baseline (speedup 1.0000x reference)
import jax
import jax.numpy as jnp
from jax.experimental import pallas as pl


def kernel(output, mask, ind, target):
    raise NotImplementedError("write your pallas kernel here")


# speedup vs baseline: 2.3822x; 2.3822x over previous
import jax
import jax.numpy as jnp
from jax import lax
from jax.experimental import pallas as pl
from jax.experimental.pallas import tpu as pltpu
from jax.experimental.pallas import tpu_sc as plsc

B, C, H, W = 16, 64, 128, 128
HW = H * W
K = 500
NW = 32
PAIRS = B * K // NW
ROWS = PAIRS * C // 128
ROWS_PAD = 128
PAIRS_PAD = ROWS_PAD * 128 // C
GRP = 8


def _sc_loss_kernel(outflat, ind2d, mask3d, targ3d, out_hbm,
                    ind_v, idx_v, pred_v, mask_v, targ_v, out_v,
                    sem_g, sem_m, sem_t):
    cid = lax.axis_index("c")
    sid = lax.axis_index("s")
    wid = sid * 2 + cid
    boff = sid * (C * HW)

    pltpu.sync_copy(ind2d.at[wid], ind_v)
    pltpu.async_copy(mask3d.at[wid], mask_v, sem_m)
    pltpu.async_copy(targ3d.at[wid], targ_v, sem_t)

    lanes = lax.iota(jnp.int32, 16)
    coffs = [lanes * HW + (q * 16 * HW) for q in range(4)]

    def build(t, _):
        ivec = ind_v[pl.ds(t * 16, 16)] + boff
        for j in range(16):
            a = ivec[j]
            row = 8 * t + j // 2
            base = (j % 2) * 64
            for q in range(4):
                idx_v[row, pl.ds(base + q * 16, 16)] = coffs[q] + a
        return 0
    lax.fori_loop(0, PAIRS_PAD // 16, build, 0)

    def fire(r):
        pltpu.async_copy(outflat.at[idx_v.at[r]], pred_v.at[r], sem_g)

    def drain(r):
        pltpu.make_async_copy(outflat.at[idx_v.at[r]], pred_v.at[r],
                              sem_g).wait()

    for j in range(GRP):
        fire(j)

    def ring(g, _):
        base = (g + 1) * GRP
        for j in range(GRP):
            fire(base + j)
        for j in range(GRP):
            drain(g * GRP + j)
        return 0
    n_grp = ROWS_PAD // GRP
    lax.fori_loop(0, n_grp - 1, ring, 0)
    for j in range(GRP):
        drain((n_grp - 1) * GRP + j)

    pltpu.make_async_copy(mask3d.at[wid], mask_v, sem_m).wait()
    pltpu.make_async_copy(targ3d.at[wid], targ_v, sem_t).wait()

    zero = jnp.zeros((16,), jnp.float32)

    def accum(r, carry):
        aabs, am = carry
        for q in range(8):
            sl = pl.ds(q * 16, 16)
            v = pred_v[r, sl]
            m = mask_v[r, sl]
            t = targ_v[r, sl]
            aabs = aabs + jnp.abs(v * m - t * m)
            am = am + m
        return (aabs, am)
    aabs, am = lax.fori_loop(0, ROWS, accum, (zero, zero))

    out_v[pl.ds(0, 16)] = aabs
    out_v[pl.ds(16, 16)] = am
    pltpu.sync_copy(out_v, out_hbm.at[wid])


@jax.jit
def kernel(output, mask, ind, target):
    outflat = output.reshape(-1)
    ind2d = jnp.pad(ind.reshape(NW, PAIRS), ((0, 0), (0, PAIRS_PAD - PAIRS)))
    mask3d = mask.reshape(NW, ROWS, 128)
    targ3d = target.reshape(NW, ROWS, 128)

    mesh = plsc.VectorSubcoreMesh(core_axis_name="c", subcore_axis_name="s")
    partials = pl.kernel(
        _sc_loss_kernel,
        mesh=mesh,
        out_type=jax.ShapeDtypeStruct((NW, 32), jnp.float32),
        scratch_types=[
            pltpu.VMEM((PAIRS_PAD,), jnp.int32),
            pltpu.VMEM((ROWS_PAD, 128), jnp.int32),
            pltpu.VMEM((ROWS_PAD, 128), jnp.float32),
            pltpu.VMEM((ROWS, 128), jnp.float32),
            pltpu.VMEM((ROWS, 128), jnp.float32),
            pltpu.VMEM((32,), jnp.float32),
            pltpu.SemaphoreType.DMA,
            pltpu.SemaphoreType.DMA,
            pltpu.SemaphoreType.DMA,
        ],
    )(outflat, ind2d, mask3d, targ3d)

    loss = jnp.sum(partials[:, :16]) / (jnp.sum(partials[:, 16:]) + 0.0001)
    return loss


# speedup vs baseline: 2.6750x; 1.1229x over previous
import jax
import jax.numpy as jnp
from jax import lax
from jax.experimental import pallas as pl
from jax.experimental.pallas import tpu as pltpu
from jax.experimental.pallas import tpu_sc as plsc

B, C, H, W = 16, 64, 128, 128
HW = H * W
K = 500
NW = 32
PAIRS = B * K // NW
ROWS = PAIRS * C // 128
ROWS_PAD = 128
PAIRS_PAD = ROWS_PAD * 128 // C
GRP = 8


def _sc_loss_kernel(outflat, ind2d, mask3d, targ3d, out_hbm,
                    ind_v, idx_v, pred_v, mask_v, targ_v, out_v,
                    sem_g, sem_m, sem_t):
    cid = lax.axis_index("c")
    sid = lax.axis_index("s")
    wid = sid * 2 + cid
    boff = sid * (C * HW)

    pltpu.sync_copy(ind2d.at[wid], ind_v)
    pltpu.async_copy(mask3d.at[wid], mask_v, sem_m)
    pltpu.async_copy(targ3d.at[wid], targ_v, sem_t)

    lanes = lax.iota(jnp.int32, 16)
    coffs = [lanes * HW + (q * 16 * HW) for q in range(4)]

    def build(t, _):
        ivec = ind_v[pl.ds(t * 16, 16)] + boff
        for j in range(16):
            a = ivec[j]
            off = (16 * t + j) * 64
            for q in range(4):
                idx_v[pl.ds(off + q * 16, 16)] = coffs[q] + a
        return 0
    lax.fori_loop(0, PAIRS_PAD // 16, build, 0)

    pltpu.async_copy(outflat.at[idx_v], pred_v, sem_g)
    pltpu.make_async_copy(outflat.at[idx_v], pred_v, sem_g).wait()

    pltpu.make_async_copy(mask3d.at[wid], mask_v, sem_m).wait()
    pltpu.make_async_copy(targ3d.at[wid], targ_v, sem_t).wait()

    zero = jnp.zeros((16,), jnp.float32)

    def accum(r, carry):
        aabs, am = carry
        for q in range(8):
            sl = pl.ds(q * 16, 16)
            v = pred_v[pl.ds(r * 128 + q * 16, 16)]
            m = mask_v[r, sl]
            t = targ_v[r, sl]
            aabs = aabs + jnp.abs(v * m - t * m)
            am = am + m
        return (aabs, am)
    aabs, am = lax.fori_loop(0, ROWS, accum, (zero, zero))

    out_v[pl.ds(0, 16)] = aabs
    out_v[pl.ds(16, 16)] = am
    pltpu.sync_copy(out_v, out_hbm.at[wid])


@jax.jit
def kernel(output, mask, ind, target):
    outflat = output.reshape(-1)
    ind2d = jnp.pad(ind.reshape(NW, PAIRS), ((0, 0), (0, PAIRS_PAD - PAIRS)))
    mask3d = mask.reshape(NW, ROWS, 128)
    targ3d = target.reshape(NW, ROWS, 128)

    mesh = plsc.VectorSubcoreMesh(core_axis_name="c", subcore_axis_name="s")
    partials = pl.kernel(
        _sc_loss_kernel,
        mesh=mesh,
        out_type=jax.ShapeDtypeStruct((NW, 32), jnp.float32),
        scratch_types=[
            pltpu.VMEM((PAIRS_PAD,), jnp.int32),
            pltpu.VMEM((ROWS_PAD * 128,), jnp.int32),
            pltpu.VMEM((ROWS_PAD * 128,), jnp.float32),
            pltpu.VMEM((ROWS, 128), jnp.float32),
            pltpu.VMEM((ROWS, 128), jnp.float32),
            pltpu.VMEM((32,), jnp.float32),
            pltpu.SemaphoreType.DMA,
            pltpu.SemaphoreType.DMA,
            pltpu.SemaphoreType.DMA,
        ],
    )(outflat, ind2d, mask3d, targ3d)

    loss = jnp.sum(partials[:, :16]) / (jnp.sum(partials[:, 16:]) + 0.0001)
    return loss
